# baseline (speedup 1.0000x reference)
import dataclasses
import functools

import jax
import jax.numpy as jnp
from jax import lax
from jax.experimental import pallas as pl
from jax.experimental.pallas import tpu as pltpu
from jax.experimental.pallas import tpu_sc as plsc

N = 10000
E = 320000
D = 128
NEG_SLOPE = 0.2

NC = 2
NS = 16
LANES = 16
HALF = N // NC
AC = 5120
EPAD = 327680
EPT = EPAD // NS
CK = 80
BLK = 2560
CPB = BLK // CK
NBLK = EPT // BLK
RPT = AC // NS
ZR = 32


def _proj_body(x_ref, w_ref, asrc_ref, adst_ref, h_ref, es_ref, ed_ref):
    h = jnp.dot(x_ref[...], w_ref[...], preferred_element_type=jnp.float32)
    h_ref[...] = h.astype(jnp.bfloat16)
    es_ref[...] = jnp.sum(h * asrc_ref[...][None, :], axis=1)
    ed_ref[...] = jnp.sum(h * adst_ref[...][None, :], axis=1)


def _final_body(acc_ref, den_ref, out_ref):
    a = acc_ref[...]
    den = jnp.sum(den_ref[0], axis=0)[:, None]
    good = den > 0.0
    val = a / jnp.where(good, den, 1.0)
    val = jnp.where(good, val, 0.0)
    out_ref[...] = jnp.where(val > 0.0, val, jnp.exp(val) - 1.0)


def _sc_body(h_hbm, es_hbm, edp_hbm, src_hbm, dst_hbm, acc_hbm, den_hbm,
             es_v, ed_v, den_v, sidx_v, didx_v, zbuf_v,
             gbuf_v, fbuf_v, row_v, sctrow_v, gidx_v, exbuf_v,
             acc_sh, gsem, ssem, sisem, disem):
    c = lax.axis_index("c")
    s = lax.axis_index("s")
    lo = c * HALF

    pltpu.sync_copy(es_hbm, es_v)
    pltpu.sync_copy(edp_hbm.at[c], ed_v)

    zero = jnp.zeros((LANES,), jnp.float32)

    @pl.loop(0, AC, step=LANES)
    def _(i):
        den_v[pl.ds(i, LANES)] = zero

    @pl.loop(0, ZR)
    def _(i):
        for v in range(D // LANES):
            zbuf_v[i, pl.ds(v * LANES, LANES)] = zero

    for k in range(RPT // ZR):
        pltpu.sync_copy(zbuf_v, acc_sh.at[pl.ds(s * RPT + k * ZR, ZR)])
    plsc.subcore_barrier()

    lo16 = jnp.full((LANES,), lo, jnp.int32)
    garbage = jnp.full((LANES,), HALF, jnp.int32) + s
    ebase = s * EPT

    def idx_start(k, t):
        off = pl.multiple_of(ebase + k * BLK, 128)
        pltpu.async_copy(src_hbm.at[pl.ds(off, BLK)],
                         sidx_v.at[t], sisem.at[t])
        pltpu.async_copy(dst_hbm.at[pl.ds(off, BLK)],
                         didx_v.at[t], disem.at[t])

    def idx_wait(t):
        pltpu.make_async_copy(src_hbm.at[pl.ds(0, BLK)],
                              sidx_v.at[t], sisem.at[t]).wait()
        pltpu.make_async_copy(dst_hbm.at[pl.ds(0, BLK)],
                              didx_v.at[t], disem.at[t]).wait()

    def gather_start(b):
        pltpu.async_copy(
            h_hbm.at[plsc.Indices(gidx_v.at[b], ignored_value=-1)],
            gbuf_v.at[b], gsem.at[b])

    def gather_wait(b):
        pltpu.make_async_copy(
            h_hbm.at[plsc.Indices(gidx_v.at[b], ignored_value=-1)],
            gbuf_v.at[b], gsem.at[b]).wait()

    def scatter_start(b):
        pltpu.async_copy(
            fbuf_v.at[b],
            acc_sh.at[plsc.Indices(sctrow_v.at[b], ignored_value=-1)],
            ssem.at[b], add=True)

    def scatter_wait(b):
        pltpu.make_async_copy(
            fbuf_v.at[b],
            acc_sh.at[plsc.Indices(sctrow_v.at[b], ignored_value=-1)],
            ssem.at[b]).wait()

    def ex_compute(ci, b, t):
        @pl.loop(0, CK, step=LANES)
        def _(i):
            s16 = sidx_v[t, pl.ds(ci * CK + i, LANES)]
            d16 = didx_v[t, pl.ds(ci * CK + i, LANES)]
            row = d16 - lo16
            local = (row >= 0) & (row < HALF)
            rowd = jnp.where(local, row, garbage)
            e = plsc.load_gather(es_v, [s16]) + plsc.load_gather(ed_v, [rowd])
            e = jnp.where(e > 0.0, e, e * NEG_SLOPE)
            ex = jnp.exp(e)
            neg1 = jnp.full((LANES,), -1, jnp.int32)
            exbuf_v[b, pl.ds(i, LANES)] = ex
            row_v[b, pl.ds(i, LANES)] = jnp.where(local, row, neg1)
            gidx_v[b, pl.ds(i, LANES)] = jnp.where(local, s16, neg1)
            plsc.addupdate_scatter(den_v, [rowd], ex)

    def scale(b):
        @pl.loop(0, CK, step=LANES)
        def _(i):
            sctrow_v[b, pl.ds(i, LANES)] = row_v[b, pl.ds(i, LANES)]

        @pl.loop(0, CK, step=LANES)
        def _(i):
            exv = exbuf_v[b, pl.ds(i, LANES)]
            for l in range(LANES):
                scv = jnp.full((LANES,), exv[l], jnp.float32)
                for v in range(D // (2 * LANES)):
                    w16 = gbuf_v[b, i + l, pl.ds(v * LANES, LANES)]
                    ab = plsc.bitcast(w16, jnp.bfloat16)
                    x0, x1 = plsc.unpack(
                        ab, format=plsc.PackFormat.INTERLEAVED)
                    fbuf_v[b, i + l, pl.ds(2 * v * LANES, LANES)] = x0 * scv
                    fbuf_v[b, i + l, pl.ds((2 * v + 1) * LANES, LANES)] = (
                        x1 * scv)

    idx_start(0, 0)

    @pl.loop(0, NBLK, step=2)
    def _(k):
        for t in range(2):
            kk = k + t
            idx_wait(t)

            @pl.when(kk + 1 < NBLK)
            def _():
                idx_start(kk + 1, 1 - t)

            ex_compute(0, 0, t)
            gather_start(0)

            @pl.loop(0, CPB, step=2)
            def _(ci, kk=kk, t=t):
                for b in range(2):
                    cur = ci + b
                    gather_wait(b)

                    @pl.when(cur + 1 < CPB)
                    def _():
                        ex_compute(cur + 1, 1 - b, t)

                    @pl.when((kk > 0) | (cur >= 1))
                    def _():
                        scatter_wait(1 - b)

                    @pl.when(cur + 1 < CPB)
                    def _():
                        gather_start(1 - b)

                    scale(b)
                    scatter_start(b)

    scatter_wait(1)
    plsc.subcore_barrier()
    pltpu.sync_copy(acc_sh.at[pl.ds(s * RPT, RPT)],
                    acc_hbm.at[c].at[pl.ds(s * RPT, RPT)])
    pltpu.sync_copy(den_v, den_hbm.at[c].at[s])


@jax.jit
def kernel(x, edge_index, W, a_src, a_dst):
    h, es, ed = pl.pallas_call(
        _proj_body,
        out_shape=(
            jax.ShapeDtypeStruct((N, D), jnp.bfloat16),
            jax.ShapeDtypeStruct((N,), jnp.float32),
            jax.ShapeDtypeStruct((N,), jnp.float32),
        ),
    )(x, W, a_src, a_dst)

    h = h.reshape(N, D // 32, 2, 16).transpose(0, 1, 3, 2)
    h = jax.lax.bitcast_convert_type(
        h.reshape(N, D // 2, 2), jnp.float32)
    src = jnp.pad(edge_index[0].astype(jnp.int32), (0, EPAD - E))
    dst = jnp.pad(edge_index[1].astype(jnp.int32), (0, EPAD - E),
                  constant_values=N)
    edp = jnp.pad(ed.reshape(NC, HALF), ((0, 0), (0, AC - HALF)))

    mesh = plsc.VectorSubcoreMesh(core_axis_name="c", subcore_axis_name="s")
    cp = pltpu.CompilerParams()
    if "needs_layout_passes" in pltpu.CompilerParams.__dataclass_fields__:
        cp = dataclasses.replace(cp, needs_layout_passes=False)
    if "use_tc_tiling_on_sc" in pltpu.CompilerParams.__dataclass_fields__:
        cp = dataclasses.replace(cp, use_tc_tiling_on_sc=False)
    acc, den = pl.kernel(
        _sc_body,
        out_type=(
            jax.ShapeDtypeStruct((NC, AC, D), jnp.float32),
            jax.ShapeDtypeStruct((NC, NS, AC), jnp.float32),
        ),
        mesh=mesh,
        compiler_params=cp,
        scratch_types=[
            pltpu.VMEM((N,), jnp.float32),
            pltpu.VMEM((AC,), jnp.float32),
            pltpu.VMEM((AC,), jnp.float32),
            pltpu.VMEM((2, BLK), jnp.int32),
            pltpu.VMEM((2, BLK), jnp.int32),
            pltpu.VMEM((ZR, D), jnp.float32),
            pltpu.VMEM((2, CK, D // 2), jnp.float32),
            pltpu.VMEM((2, CK, D), jnp.float32),
            pltpu.VMEM((2, CK), jnp.int32),
            pltpu.VMEM((2, CK), jnp.int32),
            pltpu.VMEM((2, CK), jnp.int32),
            pltpu.VMEM((2, CK), jnp.float32),
            pltpu.VMEM_SHARED((AC, D), jnp.float32),
            pltpu.SemaphoreType.DMA((2,)),
            pltpu.SemaphoreType.DMA((2,)),
            pltpu.SemaphoreType.DMA((2,)),
            pltpu.SemaphoreType.DMA((2,)),
        ],
    )(h, es, edp, src, dst)

    acc = acc[:, :HALF].reshape(N, D)
    den = jnp.concatenate([den[0, :, :HALF], den[1, :, :HALF]], axis=1)
    den = den.reshape(NS, 10, N // 10).transpose(1, 0, 2)

    out = pl.pallas_call(
        _final_body,
        grid=(10,),
        in_specs=[
            pl.BlockSpec((N // 10, D), lambda i: (i, 0)),
            pl.BlockSpec((1, NS, N // 10), lambda i: (i, 0, 0)),
        ],
        out_specs=pl.BlockSpec((N // 10, D), lambda i: (i, 0)),
        out_shape=jax.ShapeDtypeStruct((N, D), jnp.float32),
    )(acc, den)
    return out


# speedup vs baseline: 1.8333x; 1.8333x over previous
import dataclasses
import functools

import jax
import jax.numpy as jnp
from jax import lax
from jax.experimental import pallas as pl
from jax.experimental.pallas import tpu as pltpu
from jax.experimental.pallas import tpu_sc as plsc

N = 10000
E = 320000
D = 128
NEG_SLOPE = 0.2

NC = 2
NS = 16
LANES = 16
HALF = N // NC
AC = 5120
EPAD = 327680
EPT = EPAD // NS
CK = 80
BLK = 2560
CPB = BLK // CK
NBLK = EPT // BLK
RPT = AC // NS
ZR = 32


def _proj_body(x_ref, w_ref, asrc_ref, adst_ref, h_ref, es_ref, ed_ref):
    h = jnp.dot(x_ref[...], w_ref[...], preferred_element_type=jnp.float32)
    h_ref[...] = h
    es_ref[...] = jnp.sum(h * asrc_ref[...][None, :], axis=1)
    ed_ref[...] = jnp.sum(h * adst_ref[...][None, :], axis=1)


def _final_body(acc_ref, den_ref, out_ref):
    a = acc_ref[...]
    den = jnp.sum(den_ref[0], axis=0)[:, None]
    good = den > 0.0
    val = a / jnp.where(good, den, 1.0)
    val = jnp.where(good, val, 0.0)
    out_ref[...] = jnp.where(val > 0.0, val, jnp.exp(val) - 1.0)


def _sc_body(h_hbm, es_hbm, edp_hbm, src_hbm, dst_hbm, acc_hbm, den_hbm,
             es_v, ed_v, den_v, sidx_v, didx_v, zbuf_v,
             gbuf_v, row_v, sctrow_v, gidx_v, exbuf_v,
             acc_sh, gsem, ssem, sisem, disem):
    c = lax.axis_index("c")
    s = lax.axis_index("s")
    lo = c * HALF

    pltpu.sync_copy(es_hbm, es_v)
    pltpu.sync_copy(edp_hbm.at[c], ed_v)

    zero = jnp.zeros((LANES,), jnp.float32)

    @pl.loop(0, AC, step=LANES)
    def _(i):
        den_v[pl.ds(i, LANES)] = zero

    @pl.loop(0, ZR)
    def _(i):
        for v in range(D // LANES):
            zbuf_v[i, pl.ds(v * LANES, LANES)] = zero

    for k in range(RPT // ZR):
        pltpu.sync_copy(zbuf_v, acc_sh.at[pl.ds(s * RPT + k * ZR, ZR)])
    plsc.subcore_barrier()

    lo16 = jnp.full((LANES,), lo, jnp.int32)
    garbage = jnp.full((LANES,), HALF, jnp.int32) + s
    ebase = s * EPT

    def idx_start(k, t):
        off = pl.multiple_of(ebase + k * BLK, 128)
        pltpu.async_copy(src_hbm.at[pl.ds(off, BLK)],
                         sidx_v.at[t], sisem.at[t])
        pltpu.async_copy(dst_hbm.at[pl.ds(off, BLK)],
                         didx_v.at[t], disem.at[t])

    def idx_wait(t):
        pltpu.make_async_copy(src_hbm.at[pl.ds(0, BLK)],
                              sidx_v.at[t], sisem.at[t]).wait()
        pltpu.make_async_copy(dst_hbm.at[pl.ds(0, BLK)],
                              didx_v.at[t], disem.at[t]).wait()

    def gather_start(b):
        pltpu.async_copy(
            h_hbm.at[plsc.Indices(gidx_v.at[b], ignored_value=-1)],
            gbuf_v.at[b], gsem.at[b])

    def gather_wait(b):
        pltpu.make_async_copy(
            h_hbm.at[plsc.Indices(gidx_v.at[b], ignored_value=-1)],
            gbuf_v.at[b], gsem.at[b]).wait()

    def scatter_start(b):
        pltpu.async_copy(
            gbuf_v.at[b],
            acc_sh.at[plsc.Indices(sctrow_v.at[b], ignored_value=-1)],
            ssem.at[b], add=True)

    def scatter_wait(b):
        pltpu.make_async_copy(
            gbuf_v.at[b],
            acc_sh.at[plsc.Indices(sctrow_v.at[b], ignored_value=-1)],
            ssem.at[b]).wait()

    def ex_compute(ci, b, t):
        @pl.loop(0, CK, step=LANES)
        def _(i):
            s16 = sidx_v[t, pl.ds(ci * CK + i, LANES)]
            d16 = didx_v[t, pl.ds(ci * CK + i, LANES)]
            row = d16 - lo16
            local = (row >= 0) & (row < HALF)
            rowd = jnp.where(local, row, garbage)
            e = plsc.load_gather(es_v, [s16]) + plsc.load_gather(ed_v, [rowd])
            e = jnp.where(e > 0.0, e, e * NEG_SLOPE)
            ex = jnp.exp(e)
            neg1 = jnp.full((LANES,), -1, jnp.int32)
            exbuf_v[b, pl.ds(i, LANES)] = ex
            row_v[b, pl.ds(i, LANES)] = jnp.where(local, row, neg1)
            gidx_v[b, pl.ds(i, LANES)] = jnp.where(local, s16, neg1)
            plsc.addupdate_scatter(den_v, [rowd], ex)

    def scale(b):
        @pl.loop(0, CK, step=LANES)
        def _(i):
            sctrow_v[b, pl.ds(i, LANES)] = row_v[b, pl.ds(i, LANES)]

        @pl.loop(0, CK, step=LANES)
        def _(i):
            exv = exbuf_v[b, pl.ds(i, LANES)]
            for l in range(LANES):
                scv = jnp.full((LANES,), exv[l], jnp.float32)
                for v in range(D // LANES):
                    gbuf_v[b, i + l, pl.ds(v * LANES, LANES)] = (
                        gbuf_v[b, i + l, pl.ds(v * LANES, LANES)] * scv)

    idx_start(0, 0)

    @pl.loop(0, NBLK, step=2)
    def _(k):
        for t in range(2):
            kk = k + t
            idx_wait(t)

            @pl.when(kk + 1 < NBLK)
            def _():
                idx_start(kk + 1, 1 - t)

            ex_compute(0, 0, t)
            gather_start(0)

            @pl.loop(0, CPB, step=2)
            def _(ci, kk=kk, t=t):
                for b in range(2):
                    cur = ci + b
                    gather_wait(b)

                    @pl.when(cur + 1 < CPB)
                    def _():
                        ex_compute(cur + 1, 1 - b, t)

                    @pl.when((kk > 0) | (cur >= 1))
                    def _():
                        scatter_wait(1 - b)

                    @pl.when(cur + 1 < CPB)
                    def _():
                        gather_start(1 - b)

                    scale(b)
                    scatter_start(b)

    scatter_wait(1)
    plsc.subcore_barrier()
    pltpu.sync_copy(acc_sh.at[pl.ds(s * RPT, RPT)],
                    acc_hbm.at[c].at[pl.ds(s * RPT, RPT)])
    pltpu.sync_copy(den_v, den_hbm.at[c].at[s])


@jax.jit
def kernel(x, edge_index, W, a_src, a_dst):
    h, es, ed = pl.pallas_call(
        _proj_body,
        out_shape=(
            jax.ShapeDtypeStruct((N, D), jnp.float32),
            jax.ShapeDtypeStruct((N,), jnp.float32),
            jax.ShapeDtypeStruct((N,), jnp.float32),
        ),
    )(x, W, a_src, a_dst)

    src = jnp.pad(edge_index[0].astype(jnp.int32), (0, EPAD - E))
    dst = jnp.pad(edge_index[1].astype(jnp.int32), (0, EPAD - E),
                  constant_values=N)
    edp = jnp.pad(ed.reshape(NC, HALF), ((0, 0), (0, AC - HALF)))

    mesh = plsc.VectorSubcoreMesh(core_axis_name="c", subcore_axis_name="s")
    cp = pltpu.CompilerParams()
    if "needs_layout_passes" in pltpu.CompilerParams.__dataclass_fields__:
        cp = dataclasses.replace(cp, needs_layout_passes=False)
    acc, den = pl.kernel(
        _sc_body,
        out_type=(
            jax.ShapeDtypeStruct((NC, AC, D), jnp.float32),
            jax.ShapeDtypeStruct((NC, NS, AC), jnp.float32),
        ),
        mesh=mesh,
        compiler_params=cp,
        scratch_types=[
            pltpu.VMEM((N,), jnp.float32),
            pltpu.VMEM((AC,), jnp.float32),
            pltpu.VMEM((AC,), jnp.float32),
            pltpu.VMEM((2, BLK), jnp.int32),
            pltpu.VMEM((2, BLK), jnp.int32),
            pltpu.VMEM((ZR, D), jnp.float32),
            pltpu.VMEM((2, CK, D), jnp.float32),
            pltpu.VMEM((2, CK), jnp.int32),
            pltpu.VMEM((2, CK), jnp.int32),
            pltpu.VMEM((2, CK), jnp.int32),
            pltpu.VMEM((2, CK), jnp.float32),
            pltpu.VMEM_SHARED((AC, D), jnp.float32),
            pltpu.SemaphoreType.DMA((2,)),
            pltpu.SemaphoreType.DMA((2,)),
            pltpu.SemaphoreType.DMA((2,)),
            pltpu.SemaphoreType.DMA((2,)),
        ],
    )(h, es, edp, src, dst)

    acc = acc[:, :HALF].reshape(N, D)
    den = jnp.concatenate([den[0, :, :HALF], den[1, :, :HALF]], axis=1)
    den = den.reshape(NS, 10, N // 10).transpose(1, 0, 2)

    out = pl.pallas_call(
        _final_body,
        grid=(10,),
        in_specs=[
            pl.BlockSpec((N // 10, D), lambda i: (i, 0)),
            pl.BlockSpec((1, NS, N // 10), lambda i: (i, 0, 0)),
        ],
        out_specs=pl.BlockSpec((N // 10, D), lambda i: (i, 0)),
        out_shape=jax.ShapeDtypeStruct((N, D), jnp.float32),
    )(acc, den)
    return out
